# Initial kernel scaffold; baseline (speedup 1.0000x reference)
import jax
import jax.numpy as jnp
from jax.experimental import pallas as pl


def kernel(in_feat, edge_index, W1, b1, W2, b2):
    raise NotImplementedError("write your pallas kernel here")


# speedup vs baseline: 5.0660x; 5.0660x over previous
import functools

import jax
import jax.numpy as jnp
from jax import lax
from jax.experimental import pallas as pl
from jax.experimental.pallas import tpu as pltpu
from jax.experimental.pallas import tpu_sc as plsc

N_NODES = 10000
N_EDGES = 320000
NC = 2
NS = 16
N_PAD = 10240
ROWS_PER_TILE_PAD = N_PAD // NS
ROWS_PER_TILE = N_NODES // NS
K = 80

@functools.lru_cache(maxsize=None)
def _mesh():
    return plsc.VectorSubcoreMesh(core_axis_name="c", subcore_axis_name="s",
                                  num_cores=NC, num_subcores=NS)


def _deg_body(src_hbm, dst_hbm, ns_hbm, nd_hbm, acc, idxv, ones_v, degv, zv):
    c = lax.axis_index("c")
    s = lax.axis_index("s")
    one = jnp.ones((16,), jnp.float32)
    zero = jnp.zeros((16,), jnp.float32)
    for j in range(K // 16):
        ones_v[pl.ds(j * 16, 16)] = one
    for j in range(ROWS_PER_TILE_PAD // 16):
        zv[pl.ds(j * 16, 16)] = zero
    base = s * ROWS_PER_TILE_PAD
    pltpu.sync_copy(zv, acc.at[pl.ds(base, ROWS_PER_TILE_PAD)])
    plsc.subcore_barrier()

    e_per_tile = N_EDGES // NS
    nchunks = e_per_tile // K

    def chunk(i, carry):
        off = s * e_per_tile + i * K

        @pl.when(c == 0)
        def _():
            pltpu.sync_copy(src_hbm.at[pl.ds(off, K)], idxv)

        @pl.when(c != 0)
        def _():
            pltpu.sync_copy(dst_hbm.at[pl.ds(off, K)], idxv)

        pltpu.sync_copy(ones_v, acc.at[idxv], add=True)
        return carry

    lax.fori_loop(0, nchunks, chunk, 0)
    plsc.subcore_barrier()

    pltpu.sync_copy(acc.at[pl.ds(base, ROWS_PER_TILE_PAD)], degv)

    def nbody(r, carry):
        dv = degv[pl.ds(r * 16, 16)]
        d = jnp.maximum(dv, 1.0)
        i32 = lax.bitcast_convert_type(d, jnp.int32)
        y = lax.bitcast_convert_type(jnp.int32(0x5F3759DF) - (i32 >> 1),
                                     jnp.float32)
        for _ in range(3):
            y = y * (1.5 - 0.5 * d * y * y)
        degv[pl.ds(r * 16, 16)] = jnp.where(dv > 0.0, y, 0.0)
        return carry

    lax.fori_loop(0, ROWS_PER_TILE_PAD // 16, nbody, 0)

    @pl.when(c == 0)
    def _():
        pltpu.sync_copy(degv, ns_hbm.at[pl.ds(base, ROWS_PER_TILE_PAD)])

    @pl.when(c != 0)
    def _():
        pltpu.sync_copy(degv, nd_hbm.at[pl.ds(base, ROWS_PER_TILE_PAD)])


@functools.lru_cache(maxsize=None)
def _deg_call():
    return pl.kernel(
    _deg_body,
    out_type=(jax.ShapeDtypeStruct((N_PAD,), jnp.float32),
              jax.ShapeDtypeStruct((N_PAD,), jnp.float32)),
    mesh=_mesh(),
    scratch_types=[
        pltpu.VMEM_SHARED((N_PAD,), jnp.float32),
        pltpu.VMEM((K,), jnp.int32),
        pltpu.VMEM((K,), jnp.float32),
        pltpu.VMEM((ROWS_PER_TILE_PAD,), jnp.float32),
        pltpu.VMEM((ROWS_PER_TILE_PAD,), jnp.float32),
    ],
    )


def _gs_body(F, y_hbm, src_hbm, dst_hbm, out_hbm, acc, idx_s, idx_d, rows,
             zrows, sem):
    c = lax.axis_index("c")
    s = lax.axis_index("s")
    wid = c * NS + s
    zero = jnp.zeros((16,), jnp.float32)
    ZR = 128

    def zb(r, carry):
        for j in range(F // 16):
            zrows[r, pl.ds(j * 16, 16)] = zero
        return carry

    lax.fori_loop(0, ZR, zb, 0)
    base_rows = s * ROWS_PER_TILE_PAD
    for j in range(ROWS_PER_TILE_PAD // ZR):
        pltpu.sync_copy(zrows, acc.at[pl.ds(base_rows + j * ZR, ZR)])
    plsc.subcore_barrier()

    e_per_tile = N_EDGES // (NC * NS)
    nchunks = e_per_tile // K

    def chunk(i, carry):
        off = wid * e_per_tile + i * K
        pltpu.sync_copy(src_hbm.at[pl.ds(off, K)], idx_s)
        pltpu.sync_copy(dst_hbm.at[pl.ds(off, K)], idx_d)
        pltpu.async_copy(y_hbm.at[idx_s], rows, sem).wait()
        pltpu.sync_copy(rows, acc.at[idx_d], add=True)
        return carry

    lax.fori_loop(0, nchunks, chunk, 0)
    plsc.subcore_barrier()
    pltpu.sync_copy(acc.at[pl.ds(base_rows, ROWS_PER_TILE_PAD)],
                    out_hbm.at[c, pl.ds(base_rows, ROWS_PER_TILE_PAD)])


@functools.lru_cache(maxsize=None)
def _make_gs(F):
    return pl.kernel(
        functools.partial(_gs_body, F),
        out_type=jax.ShapeDtypeStruct((NC, N_PAD, F), jnp.float32),
        mesh=_mesh(),
        scratch_types=[
            pltpu.VMEM_SHARED((N_PAD, F), jnp.float32),
            pltpu.VMEM((K,), jnp.int32),
            pltpu.VMEM((K,), jnp.int32),
            pltpu.VMEM((K, F), jnp.float32),
            pltpu.VMEM((128, F), jnp.float32),
            pltpu.SemaphoreType.DMA,
        ],
        compiler_params=pltpu.CompilerParams(
            use_tc_tiling_on_sc=(F % 128 == 0)),
    )


def _tc1_body(x_ref, ns_ref, w_ref, o_ref):
    o_ref[...] = jnp.dot(x_ref[...] * ns_ref[...], w_ref[...],
                         preferred_element_type=jnp.float32)


def _tc2_body(p_ref, nd_ref, b1_ref, ns_ref, w2_ref, o_ref):
    agg = p_ref[0, :N_NODES] + p_ref[1, :N_NODES]
    h = jnp.maximum(agg * nd_ref[...] + b1_ref[...], 0.0)
    o_ref[...] = jnp.dot(h * ns_ref[...], w2_ref[...],
                         preferred_element_type=jnp.float32)


def _tc3_body(p_ref, nd_ref, b2_ref, o_ref):
    agg = p_ref[0, :N_NODES] + p_ref[1, :N_NODES]
    o_ref[...] = jnp.maximum(agg * nd_ref[...] + b2_ref[...], 0.0)


def _tc1_call(x, ns, w1):
    return pl.pallas_call(
        _tc1_body,
        out_shape=jax.ShapeDtypeStruct((N_NODES, w1.shape[1]), jnp.float32),
    )(x, ns, w1)


def _tc2_call(p, nd, b1, ns, w2):
    return pl.pallas_call(
        _tc2_body,
        out_shape=jax.ShapeDtypeStruct((N_NODES, w2.shape[1]), jnp.float32),
    )(p, nd, b1, ns, w2)


def _tc3_call(p, nd, b2):
    return pl.pallas_call(
        _tc3_body,
        out_shape=jax.ShapeDtypeStruct((N_NODES, p.shape[2]), jnp.float32),
    )(p, nd, b2)


def kernel(in_feat, edge_index, W1, b1, W2, b2):
    ei = edge_index.astype(jnp.int32)
    src = ei[0]
    dst = ei[1]
    ns_pad, nd_pad = _deg_call()(src, dst)
    ns = ns_pad[:N_NODES].reshape(N_NODES, 1)
    nd = nd_pad[:N_NODES].reshape(N_NODES, 1)
    y1 = _tc1_call(in_feat, ns, W1)
    p1 = _make_gs(128)(y1, src, dst)
    y2 = _tc2_call(p1, nd, b1, ns, W2)
    p2 = _make_gs(64)(y2, src, dst)
    return _tc3_call(p2, nd, b2)


# speedup vs baseline: 7.4583x; 1.4722x over previous
import functools

import jax
import jax.numpy as jnp
from jax import lax
from jax.experimental import pallas as pl
from jax.experimental.pallas import tpu as pltpu
from jax.experimental.pallas import tpu_sc as plsc

N_NODES = 10000
N_EDGES = 320000
NC = 2
NS = 16
N_PAD = 10240
ROWS_PER_TILE_PAD = N_PAD // NS
CK = 100
DCH = N_EDGES // NS // CK
CKP = 128
ECH = 158


@functools.lru_cache(maxsize=None)
def _mesh():
    return plsc.VectorSubcoreMesh(core_axis_name="c", subcore_axis_name="s",
                                  num_cores=NC, num_subcores=NS)


def _deg_body(src_hbm, dst_hbm, ns_hbm, nd_hbm, acc, idxm, ones_v, degv, zv,
              sm0, sm1, sm2, sm3):
    c = lax.axis_index("c")
    s = lax.axis_index("s")
    one = jnp.ones((16,), jnp.float32)
    zero = jnp.zeros((16,), jnp.float32)
    for j in range(112 // 16):
        ones_v[pl.ds(j * 16, 16)] = one
    for j in range(ROWS_PER_TILE_PAD // 16):
        zv[pl.ds(j * 16, 16)] = zero
    base = s * ROWS_PER_TILE_PAD
    pltpu.sync_copy(zv, acc.at[pl.ds(base, ROWS_PER_TILE_PAD)])

    @pl.when(c == 0)
    def _():
        pltpu.sync_copy(src_hbm.at[s], idxm)

    @pl.when(c != 0)
    def _():
        pltpu.sync_copy(dst_hbm.at[s], idxm)

    plsc.subcore_barrier()

    ones_c = ones_v.at[pl.ds(0, CK)]
    sems = (sm0, sm1, sm2, sm3)
    for b in range(4):
        pltpu.async_copy(ones_c, acc.at[idxm.at[b]], sems[b], add=True)

    def ring(k, carry):
        i = 4 * k
        for b in range(4):
            pltpu.make_async_copy(ones_c, acc.at[idxm.at[i + b]],
                                  sems[b]).wait()

            @pl.when(k + 1 < DCH // 4)
            def _():
                pltpu.async_copy(ones_c, acc.at[idxm.at[i + 4 + b]], sems[b],
                                 add=True)

        return carry

    lax.fori_loop(0, DCH // 4, ring, 0)
    plsc.subcore_barrier()

    pltpu.sync_copy(acc.at[pl.ds(base, ROWS_PER_TILE_PAD)], degv)

    def nbody(r, carry):
        dv = degv[pl.ds(r * 16, 16)]
        d = jnp.maximum(dv, 1.0)
        i32 = lax.bitcast_convert_type(d, jnp.int32)
        y = lax.bitcast_convert_type(jnp.int32(0x5F3759DF) - (i32 >> 1),
                                     jnp.float32)
        for _ in range(3):
            y = y * (1.5 - 0.5 * d * y * y)
        degv[pl.ds(r * 16, 16)] = jnp.where(dv > 0.0, y, 0.0)
        return carry

    lax.fori_loop(0, ROWS_PER_TILE_PAD // 16, nbody, 0)

    @pl.when(c == 0)
    def _():
        pltpu.sync_copy(degv, ns_hbm.at[pl.ds(base, ROWS_PER_TILE_PAD)])

    @pl.when(c != 0)
    def _():
        pltpu.sync_copy(degv, nd_hbm.at[pl.ds(base, ROWS_PER_TILE_PAD)])


@functools.lru_cache(maxsize=None)
def _deg_call():
    return pl.kernel(
        _deg_body,
        out_type=(jax.ShapeDtypeStruct((N_PAD,), jnp.float32),
                  jax.ShapeDtypeStruct((N_PAD,), jnp.float32)),
        mesh=_mesh(),
        scratch_types=[
            pltpu.VMEM_SHARED((N_PAD,), jnp.float32),
            pltpu.VMEM((DCH, CK), jnp.int32),
            pltpu.VMEM((112,), jnp.float32),
            pltpu.VMEM((ROWS_PER_TILE_PAD,), jnp.float32),
            pltpu.VMEM((ROWS_PER_TILE_PAD,), jnp.float32),
            pltpu.SemaphoreType.DMA,
            pltpu.SemaphoreType.DMA,
            pltpu.SemaphoreType.DMA,
            pltpu.SemaphoreType.DMA,
        ],
    )


def _gsh_body(FH, yr_hbm, srcx_hbm, dstx_hbm, out_hbm, acc, idx_s, idx_d,
              rows0, rows1, sg0, sg1):
    c = lax.axis_index("c")
    s = lax.axis_index("s")
    zero = jnp.zeros((16,), jnp.float32)

    def zb(r, carry):
        for j in range(FH // 16):
            rows0[r, pl.ds(j * 16, 16)] = zero
        return carry

    lax.fori_loop(0, CKP, zb, 0)
    base_rows = s * ROWS_PER_TILE_PAD
    for j in range(ROWS_PER_TILE_PAD // CKP):
        pltpu.sync_copy(rows0, acc.at[pl.ds(base_rows + j * CKP, CKP)])
    pltpu.sync_copy(srcx_hbm.at[c, s], idx_s)
    pltpu.sync_copy(dstx_hbm.at[s], idx_d)
    plsc.subcore_barrier()

    pltpu.async_copy(yr_hbm.at[idx_s.at[0]], rows0, sg0)
    NP = ECH // 2

    def pair(k, carry):
        i = 2 * k
        pltpu.async_copy(yr_hbm.at[idx_s.at[i + 1]], rows1, sg1)
        pltpu.make_async_copy(yr_hbm.at[idx_s.at[i]], rows0, sg0).wait()
        pltpu.sync_copy(rows0, acc.at[idx_d.at[i]], add=True)

        @pl.when(k + 1 < NP)
        def _():
            pltpu.async_copy(yr_hbm.at[idx_s.at[i + 2]], rows0, sg0)

        pltpu.make_async_copy(yr_hbm.at[idx_s.at[i + 1]], rows1, sg1).wait()
        pltpu.sync_copy(rows1, acc.at[idx_d.at[i + 1]], add=True)
        return carry

    lax.fori_loop(0, NP, pair, 0)
    plsc.subcore_barrier()
    pltpu.sync_copy(acc.at[pl.ds(base_rows, ROWS_PER_TILE_PAD)],
                    out_hbm.at[c, pl.ds(base_rows, ROWS_PER_TILE_PAD)])


@functools.lru_cache(maxsize=None)
def _make_gsh(FH):
    return pl.kernel(
        functools.partial(_gsh_body, FH),
        out_type=jax.ShapeDtypeStruct((NC, N_PAD, FH), jnp.float32),
        mesh=_mesh(),
        scratch_types=[
            pltpu.VMEM_SHARED((N_PAD, FH), jnp.float32),
            pltpu.VMEM((ECH, CKP), jnp.int32),
            pltpu.VMEM((ECH, CKP), jnp.int32),
            pltpu.VMEM((CKP, FH), jnp.float32),
            pltpu.VMEM((CKP, FH), jnp.float32),
            pltpu.SemaphoreType.DMA,
            pltpu.SemaphoreType.DMA,
        ],
        compiler_params=pltpu.CompilerParams(use_tc_tiling_on_sc=False),
    )


def _prep_idx(src, dst):
    e_tile = N_EDGES // NS
    padt = ECH * CKP - e_tile
    s2 = (src * 2).reshape(NS, e_tile)
    zpad = jnp.zeros((NS, padt), jnp.int32)
    lo = jnp.concatenate([s2, zpad], axis=1)
    hi = jnp.concatenate([s2 + 1, zpad], axis=1)
    srcx = jnp.stack([lo, hi]).reshape(NC, NS, ECH, CKP)
    dpad = jnp.full((NS, padt), N_NODES, jnp.int32)
    dstx = jnp.concatenate([dst.reshape(NS, e_tile), dpad],
                           axis=1).reshape(NS, ECH, CKP)
    return srcx, dstx


def _tc1_body(x_ref, ns_ref, w_ref, o_ref):
    o_ref[...] = jnp.dot(x_ref[...] * ns_ref[...], w_ref[...],
                         preferred_element_type=jnp.float32)


def _tc2_body(p_ref, nd_ref, b1_ref, ns_ref, w2_ref, o_ref):
    agg = jnp.concatenate([p_ref[0, :N_NODES], p_ref[1, :N_NODES]], axis=1)
    h = jnp.maximum(agg * nd_ref[...] + b1_ref[...], 0.0)
    o_ref[...] = jnp.dot(h * ns_ref[...], w2_ref[...],
                         preferred_element_type=jnp.float32)


def _tc3_body(p_ref, nd_ref, b2_ref, o_ref):
    agg = jnp.concatenate([p_ref[0, :N_NODES], p_ref[1, :N_NODES]], axis=1)
    o_ref[...] = jnp.maximum(agg * nd_ref[...] + b2_ref[...], 0.0)


def _tc1_call(x, ns, w1):
    return pl.pallas_call(
        _tc1_body,
        out_shape=jax.ShapeDtypeStruct((N_NODES, w1.shape[1]), jnp.float32),
    )(x, ns, w1)


def _tc2_call(p, nd, b1, ns, w2):
    return pl.pallas_call(
        _tc2_body,
        out_shape=jax.ShapeDtypeStruct((N_NODES, w2.shape[1]), jnp.float32),
    )(p, nd, b1, ns, w2)


def _tc3_call(p, nd, b2):
    return pl.pallas_call(
        _tc3_body,
        out_shape=jax.ShapeDtypeStruct((N_NODES, 2 * p.shape[2]), jnp.float32),
    )(p, nd, b2)


def kernel(in_feat, edge_index, W1, b1, W2, b2):
    ei = edge_index.astype(jnp.int32)
    src = ei[0]
    dst = ei[1]
    srcd = src.reshape(NS, DCH, CK)
    dstd = dst.reshape(NS, DCH, CK)
    srcx, dstx = _prep_idx(src, dst)
    ns_pad, nd_pad = _deg_call()(srcd, dstd)
    ns = ns_pad[:N_NODES].reshape(N_NODES, 1)
    nd = nd_pad[:N_NODES].reshape(N_NODES, 1)
    y1 = _tc1_call(in_feat, ns, W1)
    p1 = _make_gsh(64)(y1.reshape(2 * N_NODES, 64), srcx, dstx)
    y2 = _tc2_call(p1, nd, b1, ns, W2)
    p2 = _make_gsh(32)(y2.reshape(2 * N_NODES, 32), srcx, dstx)
    return _tc3_call(p2, nd, b2)


# speedup vs baseline: 10.8976x; 1.4611x over previous
import functools

import jax
import jax.numpy as jnp
from jax import lax
from jax.experimental import pallas as pl
from jax.experimental.pallas import tpu as pltpu
from jax.experimental.pallas import tpu_sc as plsc

N_NODES = 10000
N_EDGES = 320000
NC = 2
NS = 16
N_PAD = 10240
ROWS_PER_TILE_PAD = N_PAD // NS
CK = 100
DCH = N_EDGES // NS // CK
CKP = 128
ECH = 160


@functools.lru_cache(maxsize=None)
def _mesh():
    return plsc.VectorSubcoreMesh(core_axis_name="c", subcore_axis_name="s",
                                  num_cores=NC, num_subcores=NS)


def _deg_body(src_hbm, dst_hbm, ns_hbm, nd_hbm, acc, idxm, ones_v, degv, zv,
              sm0, sm1, sm2, sm3):
    c = lax.axis_index("c")
    s = lax.axis_index("s")
    one = jnp.ones((16,), jnp.float32)
    zero = jnp.zeros((16,), jnp.float32)
    for j in range(112 // 16):
        ones_v[pl.ds(j * 16, 16)] = one
    for j in range(ROWS_PER_TILE_PAD // 16):
        zv[pl.ds(j * 16, 16)] = zero
    base = s * ROWS_PER_TILE_PAD
    pltpu.sync_copy(zv, acc.at[pl.ds(base, ROWS_PER_TILE_PAD)])

    @pl.when(c == 0)
    def _():
        pltpu.sync_copy(src_hbm.at[s], idxm)

    @pl.when(c != 0)
    def _():
        pltpu.sync_copy(dst_hbm.at[s], idxm)

    plsc.subcore_barrier()

    ones_c = ones_v.at[pl.ds(0, CK)]
    sems = (sm0, sm1, sm2, sm3)
    for b in range(4):
        pltpu.async_copy(ones_c, acc.at[idxm.at[b]], sems[b], add=True)

    def ring(k, carry):
        i = 4 * k
        for b in range(4):
            pltpu.make_async_copy(ones_c, acc.at[idxm.at[i + b]],
                                  sems[b]).wait()

            @pl.when(k + 1 < DCH // 4)
            def _():
                pltpu.async_copy(ones_c, acc.at[idxm.at[i + 4 + b]], sems[b],
                                 add=True)

        return carry

    lax.fori_loop(0, DCH // 4, ring, 0)
    plsc.subcore_barrier()

    pltpu.sync_copy(acc.at[pl.ds(base, ROWS_PER_TILE_PAD)], degv)

    def nbody(r, carry):
        dv = degv[pl.ds(r * 16, 16)]
        d = jnp.maximum(dv, 1.0)
        i32 = lax.bitcast_convert_type(d, jnp.int32)
        y = lax.bitcast_convert_type(jnp.int32(0x5F3759DF) - (i32 >> 1),
                                     jnp.float32)
        for _ in range(3):
            y = y * (1.5 - 0.5 * d * y * y)
        degv[pl.ds(r * 16, 16)] = jnp.where(dv > 0.0, y, 0.0)
        return carry

    lax.fori_loop(0, ROWS_PER_TILE_PAD // 16, nbody, 0)

    @pl.when(c == 0)
    def _():
        pltpu.sync_copy(degv, ns_hbm.at[pl.ds(base, ROWS_PER_TILE_PAD)])

    @pl.when(c != 0)
    def _():
        pltpu.sync_copy(degv, nd_hbm.at[pl.ds(base, ROWS_PER_TILE_PAD)])


@functools.lru_cache(maxsize=None)
def _deg_call():
    return pl.kernel(
        _deg_body,
        out_type=(jax.ShapeDtypeStruct((N_PAD,), jnp.float32),
                  jax.ShapeDtypeStruct((N_PAD,), jnp.float32)),
        mesh=_mesh(),
        scratch_types=[
            pltpu.VMEM_SHARED((N_PAD,), jnp.float32),
            pltpu.VMEM((DCH, CK), jnp.int32),
            pltpu.VMEM((112,), jnp.float32),
            pltpu.VMEM((ROWS_PER_TILE_PAD,), jnp.float32),
            pltpu.VMEM((ROWS_PER_TILE_PAD,), jnp.float32),
            pltpu.SemaphoreType.DMA,
            pltpu.SemaphoreType.DMA,
            pltpu.SemaphoreType.DMA,
            pltpu.SemaphoreType.DMA,
        ],
    )


def _gsh_body(FH, idx_halves, yh_hbm, src_hbm, dst_hbm, out_hbm, acc, ytab,
              idx_s, idx_d, rows0, rows1, sg0, sg1):
    c = lax.axis_index("c")
    s = lax.axis_index("s")
    zero = jnp.zeros((16,), jnp.float32)

    def zb(r, carry):
        for j in range(FH // 16):
            rows0[r, pl.ds(j * 16, 16)] = zero
        return carry

    lax.fori_loop(0, CKP, zb, 0)
    base_rows = s * ROWS_PER_TILE_PAD
    for j in range(ROWS_PER_TILE_PAD // CKP):
        pltpu.sync_copy(rows0, acc.at[pl.ds(base_rows + j * CKP, CKP)])
    tslice = N_NODES // NS
    pltpu.sync_copy(yh_hbm.at[c, pl.ds(s * tslice, tslice)],
                    ytab.at[pl.ds(s * tslice, tslice)])
    nh = ECH // idx_halves

    def run_half(h):
        pltpu.sync_copy(src_hbm.at[s, pl.ds(h * nh, nh)], idx_s)
        pltpu.sync_copy(dst_hbm.at[s, pl.ds(h * nh, nh)], idx_d)
        if h == 0:
            plsc.subcore_barrier()
        pltpu.async_copy(ytab.at[idx_s.at[0]], rows0, sg0)

        def pair(k, carry):
            i = 2 * k
            pltpu.async_copy(ytab.at[idx_s.at[i + 1]], rows1, sg1)
            pltpu.make_async_copy(ytab.at[idx_s.at[i]], rows0, sg0).wait()
            pltpu.sync_copy(rows0, acc.at[idx_d.at[i]], add=True)

            @pl.when(k + 1 < nh // 2)
            def _():
                pltpu.async_copy(ytab.at[idx_s.at[i + 2]], rows0, sg0)

            pltpu.make_async_copy(ytab.at[idx_s.at[i + 1]], rows1, sg1).wait()
            pltpu.sync_copy(rows1, acc.at[idx_d.at[i + 1]], add=True)
            return carry

        lax.fori_loop(0, nh // 2, pair, 0)

    for h in range(idx_halves):
        run_half(h)
    plsc.subcore_barrier()
    pltpu.sync_copy(acc.at[pl.ds(base_rows, ROWS_PER_TILE_PAD)],
                    out_hbm.at[c, pl.ds(base_rows, ROWS_PER_TILE_PAD)])


@functools.lru_cache(maxsize=None)
def _make_gsh(FH, idx_halves):
    return pl.kernel(
        functools.partial(_gsh_body, FH, idx_halves),
        out_type=jax.ShapeDtypeStruct((NC, N_PAD, FH), jnp.float32),
        mesh=_mesh(),
        scratch_types=[
            pltpu.VMEM_SHARED((N_PAD, FH), jnp.float32),
            pltpu.VMEM_SHARED((N_NODES, FH), jnp.float32),
            pltpu.VMEM((ECH // idx_halves, CKP), jnp.int32),
            pltpu.VMEM((ECH // idx_halves, CKP), jnp.int32),
            pltpu.VMEM((CKP, FH), jnp.float32),
            pltpu.VMEM((CKP, FH), jnp.float32),
            pltpu.SemaphoreType.DMA,
            pltpu.SemaphoreType.DMA,
        ],
        compiler_params=pltpu.CompilerParams(use_tc_tiling_on_sc=False),
    )


def _prep_idx(src, dst):
    e_tile = N_EDGES // NS
    padt = ECH * CKP - e_tile
    zpad = jnp.zeros((NS, padt), jnp.int32)
    srcp = jnp.concatenate([src.reshape(NS, e_tile), zpad],
                           axis=1).reshape(NS, ECH, CKP)
    dpad = jnp.full((NS, padt), N_NODES, jnp.int32)
    dstp = jnp.concatenate([dst.reshape(NS, e_tile), dpad],
                           axis=1).reshape(NS, ECH, CKP)
    return srcp, dstp


def _tc1_body(x_ref, ns_ref, w_ref, o_ref):
    z = jnp.dot(x_ref[...] * ns_ref[...], w_ref[...],
                preferred_element_type=jnp.float32)
    fh = z.shape[1] // 2
    o_ref[0] = z[:, :fh]
    o_ref[1] = z[:, fh:]


def _tc2_body(p_ref, nd_ref, b1_ref, ns_ref, w2_ref, o_ref):
    agg = jnp.concatenate([p_ref[0, :N_NODES], p_ref[1, :N_NODES]], axis=1)
    h = jnp.maximum(agg * nd_ref[...] + b1_ref[...], 0.0)
    z = jnp.dot(h * ns_ref[...], w2_ref[...],
                preferred_element_type=jnp.float32)
    fh = z.shape[1] // 2
    o_ref[0] = z[:, :fh]
    o_ref[1] = z[:, fh:]


def _tc3_body(p_ref, nd_ref, b2_ref, o_ref):
    agg = jnp.concatenate([p_ref[0, :N_NODES], p_ref[1, :N_NODES]], axis=1)
    o_ref[...] = jnp.maximum(agg * nd_ref[...] + b2_ref[...], 0.0)


def _tc1_call(x, ns, w1):
    return pl.pallas_call(
        _tc1_body,
        out_shape=jax.ShapeDtypeStruct((2, N_NODES, w1.shape[1] // 2),
                                       jnp.float32),
    )(x, ns, w1)


def _tc2_call(p, nd, b1, ns, w2):
    return pl.pallas_call(
        _tc2_body,
        out_shape=jax.ShapeDtypeStruct((2, N_NODES, w2.shape[1] // 2),
                                       jnp.float32),
    )(p, nd, b1, ns, w2)


def _tc3_call(p, nd, b2):
    return pl.pallas_call(
        _tc3_body,
        out_shape=jax.ShapeDtypeStruct((N_NODES, 2 * p.shape[2]), jnp.float32),
    )(p, nd, b2)


def kernel(in_feat, edge_index, W1, b1, W2, b2):
    ei = edge_index.astype(jnp.int32)
    src = ei[0]
    dst = ei[1]
    srcd = src.reshape(NS, DCH, CK)
    dstd = dst.reshape(NS, DCH, CK)
    srcx, dstx = _prep_idx(src, dst)
    ns_pad, nd_pad = _deg_call()(srcd, dstd)
    ns = ns_pad[:N_NODES].reshape(N_NODES, 1)
    nd = nd_pad[:N_NODES].reshape(N_NODES, 1)
    y1h = _tc1_call(in_feat, ns, W1)
    p1 = _make_gsh(64, 2)(y1h, srcx, dstx)
    y2h = _tc2_call(p1, nd, b1, ns, W2)
    p2 = _make_gsh(32, 1)(y2h, srcx, dstx)
    return _tc3_call(p2, nd, b2)


# speedup vs baseline: 11.0211x; 1.0113x over previous
import functools

import jax
import jax.numpy as jnp
from jax import lax
from jax.experimental import pallas as pl
from jax.experimental.pallas import tpu as pltpu
from jax.experimental.pallas import tpu_sc as plsc

N_NODES = 10000
N_EDGES = 320000
NC = 2
NS = 16
N_PAD = 10240
ROWS_PER_TILE_PAD = N_PAD // NS
CK = 100
DCH = N_EDGES // NS // CK
CKP = 128
ECH = 160


@functools.lru_cache(maxsize=None)
def _mesh():
    return plsc.VectorSubcoreMesh(core_axis_name="c", subcore_axis_name="s",
                                  num_cores=NC, num_subcores=NS)


def _deg_body(src_hbm, dst_hbm, ns_hbm, nd_hbm, acc, idxm, ones_v, degv, zv,
              sm0, sm1, sm2, sm3):
    c = lax.axis_index("c")
    s = lax.axis_index("s")
    one = jnp.ones((16,), jnp.float32)
    zero = jnp.zeros((16,), jnp.float32)
    for j in range(112 // 16):
        ones_v[pl.ds(j * 16, 16)] = one
    for j in range(ROWS_PER_TILE_PAD // 16):
        zv[pl.ds(j * 16, 16)] = zero
    base = s * ROWS_PER_TILE_PAD
    pltpu.sync_copy(zv, acc.at[pl.ds(base, ROWS_PER_TILE_PAD)])

    @pl.when(c == 0)
    def _():
        pltpu.sync_copy(src_hbm.at[s], idxm)

    @pl.when(c != 0)
    def _():
        pltpu.sync_copy(dst_hbm.at[s], idxm)

    plsc.subcore_barrier()

    ones_c = ones_v.at[pl.ds(0, CK)]
    sems = (sm0, sm1, sm2, sm3)
    for b in range(4):
        pltpu.async_copy(ones_c, acc.at[idxm.at[b]], sems[b], add=True)

    def ring(k, carry):
        i = 4 * k
        for b in range(4):
            pltpu.make_async_copy(ones_c, acc.at[idxm.at[i + b]],
                                  sems[b]).wait()

            @pl.when(k + 1 < DCH // 4)
            def _():
                pltpu.async_copy(ones_c, acc.at[idxm.at[i + 4 + b]], sems[b],
                                 add=True)

        return carry

    lax.fori_loop(0, DCH // 4, ring, 0)
    plsc.subcore_barrier()

    pltpu.sync_copy(acc.at[pl.ds(base, ROWS_PER_TILE_PAD)], degv)

    def nbody(r, carry):
        dv = degv[pl.ds(r * 16, 16)]
        d = jnp.maximum(dv, 1.0)
        i32 = lax.bitcast_convert_type(d, jnp.int32)
        y = lax.bitcast_convert_type(jnp.int32(0x5F3759DF) - (i32 >> 1),
                                     jnp.float32)
        for _ in range(3):
            y = y * (1.5 - 0.5 * d * y * y)
        degv[pl.ds(r * 16, 16)] = jnp.where(dv > 0.0, y, 0.0)
        return carry

    lax.fori_loop(0, ROWS_PER_TILE_PAD // 16, nbody, 0)

    @pl.when(c == 0)
    def _():
        pltpu.sync_copy(degv, ns_hbm.at[pl.ds(base, ROWS_PER_TILE_PAD)])

    @pl.when(c != 0)
    def _():
        pltpu.sync_copy(degv, nd_hbm.at[pl.ds(base, ROWS_PER_TILE_PAD)])


@functools.lru_cache(maxsize=None)
def _deg_call():
    return pl.kernel(
        _deg_body,
        out_type=(jax.ShapeDtypeStruct((N_PAD,), jnp.float32),
                  jax.ShapeDtypeStruct((N_PAD,), jnp.float32)),
        mesh=_mesh(),
        scratch_types=[
            pltpu.VMEM_SHARED((N_PAD,), jnp.float32),
            pltpu.VMEM((DCH, CK), jnp.int32),
            pltpu.VMEM((112,), jnp.float32),
            pltpu.VMEM((ROWS_PER_TILE_PAD,), jnp.float32),
            pltpu.VMEM((ROWS_PER_TILE_PAD,), jnp.float32),
            pltpu.SemaphoreType.DMA,
            pltpu.SemaphoreType.DMA,
            pltpu.SemaphoreType.DMA,
            pltpu.SemaphoreType.DMA,
        ],
    )


def _gsh_body(FH, idx_halves, finalize, *refs):
    if finalize:
        (yh_hbm, src_hbm, dst_hbm, nd_hbm, b_hbm, out_hbm, acc, ytab,
         idx_s, idx_d, rows0, rows1, ndv, bv, sg0, sg1) = refs
    else:
        (yh_hbm, src_hbm, dst_hbm, out_hbm, acc, ytab,
         idx_s, idx_d, rows0, rows1, sg0, sg1) = refs
    c = lax.axis_index("c")
    s = lax.axis_index("s")
    zero = jnp.zeros((16,), jnp.float32)

    def zb(r, carry):
        for j in range(FH // 16):
            rows0[r, pl.ds(j * 16, 16)] = zero
        return carry

    lax.fori_loop(0, CKP, zb, 0)
    base_rows = s * ROWS_PER_TILE_PAD
    for j in range(ROWS_PER_TILE_PAD // CKP):
        pltpu.sync_copy(rows0, acc.at[pl.ds(base_rows + j * CKP, CKP)])
    tslice = N_NODES // NS
    pltpu.sync_copy(yh_hbm.at[c, pl.ds(s * tslice, tslice)],
                    ytab.at[pl.ds(s * tslice, tslice)])
    nh = ECH // idx_halves

    def run_half(h):
        pltpu.sync_copy(src_hbm.at[s, pl.ds(h * nh, nh)], idx_s)
        pltpu.sync_copy(dst_hbm.at[s, pl.ds(h * nh, nh)], idx_d)
        if h == 0:
            plsc.subcore_barrier()
        pltpu.async_copy(ytab.at[idx_s.at[0]], rows0, sg0)

        def pair(k, carry):
            i = 2 * k
            pltpu.async_copy(ytab.at[idx_s.at[i + 1]], rows1, sg1)
            pltpu.make_async_copy(ytab.at[idx_s.at[i]], rows0, sg0).wait()
            pltpu.sync_copy(rows0, acc.at[idx_d.at[i]], add=True)

            @pl.when(k + 1 < nh // 2)
            def _():
                pltpu.async_copy(ytab.at[idx_s.at[i + 2]], rows0, sg0)

            pltpu.make_async_copy(ytab.at[idx_s.at[i + 1]], rows1, sg1).wait()
            pltpu.sync_copy(rows1, acc.at[idx_d.at[i + 1]], add=True)
            return carry

        lax.fori_loop(0, nh // 2, pair, 0)

    for h in range(idx_halves):
        run_half(h)
    plsc.subcore_barrier()
    if not finalize:
        pltpu.sync_copy(acc.at[pl.ds(base_rows, ROWS_PER_TILE_PAD)],
                        out_hbm.at[c, pl.ds(base_rows, ROWS_PER_TILE_PAD)])
        return

    pltpu.sync_copy(b_hbm.at[pl.ds(c * FH, FH)], bv)
    for blk in range(ROWS_PER_TILE_PAD // CKP):
        rb = base_rows + blk * CKP
        pltpu.sync_copy(acc.at[pl.ds(rb, CKP)], rows0)
        pltpu.sync_copy(nd_hbm.at[pl.ds(rb, CKP)], ndv)

        def rowp(r16, carry):
            ndr16 = ndv[pl.ds(r16 * 16, 16)]
            for rr in range(16):
                r = r16 * 16 + rr
                nd_s = ndr16[rr]
                for j in range(FH // 16):
                    v = rows0[r, pl.ds(j * 16, 16)]
                    rows0[r, pl.ds(j * 16, 16)] = jnp.maximum(
                        v * nd_s + bv[pl.ds(j * 16, 16)], 0.0)
            return carry

        lax.fori_loop(0, CKP // 16, rowp, 0)
        rem = N_NODES % CKP

        @pl.when(rb + CKP <= N_NODES)
        def _():
            pltpu.sync_copy(rows0,
                            out_hbm.at[pl.ds(rb, CKP), pl.ds(c * FH, FH)])

        @pl.when(jnp.logical_and(rb < N_NODES, rb + CKP > N_NODES))
        def _():
            pltpu.sync_copy(rows0.at[pl.ds(0, rem)],
                            out_hbm.at[pl.ds(rb, rem), pl.ds(c * FH, FH)])


@functools.lru_cache(maxsize=None)
def _make_gsh(FH, idx_halves, finalize=False):
    if finalize:
        out_type = jax.ShapeDtypeStruct((N_NODES, 2 * FH), jnp.float32)
        extra = [pltpu.VMEM((CKP,), jnp.float32),
                 pltpu.VMEM((FH,), jnp.float32)]
    else:
        out_type = jax.ShapeDtypeStruct((NC, N_PAD, FH), jnp.float32)
        extra = []
    return pl.kernel(
        functools.partial(_gsh_body, FH, idx_halves, finalize),
        out_type=out_type,
        mesh=_mesh(),
        scratch_types=[
            pltpu.VMEM_SHARED((N_PAD, FH), jnp.float32),
            pltpu.VMEM_SHARED((N_NODES, FH), jnp.float32),
            pltpu.VMEM((ECH // idx_halves, CKP), jnp.int32),
            pltpu.VMEM((ECH // idx_halves, CKP), jnp.int32),
            pltpu.VMEM((CKP, FH), jnp.float32),
            pltpu.VMEM((CKP, FH), jnp.float32),
        ] + extra + [
            pltpu.SemaphoreType.DMA,
            pltpu.SemaphoreType.DMA,
        ],
        compiler_params=pltpu.CompilerParams(use_tc_tiling_on_sc=False),
    )


def _prep_idx(src, dst):
    e_tile = N_EDGES // NS
    padt = ECH * CKP - e_tile
    zpad = jnp.zeros((NS, padt), jnp.int32)
    srcp = jnp.concatenate([src.reshape(NS, e_tile), zpad],
                           axis=1).reshape(NS, ECH, CKP)
    dpad = jnp.full((NS, padt), N_NODES, jnp.int32)
    dstp = jnp.concatenate([dst.reshape(NS, e_tile), dpad],
                           axis=1).reshape(NS, ECH, CKP)
    return srcp, dstp


def _tc1_body(x_ref, ns_ref, w_ref, o_ref):
    z = jnp.dot(x_ref[...] * ns_ref[...], w_ref[...],
                preferred_element_type=jnp.float32)
    fh = z.shape[1] // 2
    o_ref[0] = z[:, :fh]
    o_ref[1] = z[:, fh:]


def _tc2_body(p_ref, nd_ref, b1_ref, ns_ref, w2_ref, o_ref):
    agg = jnp.concatenate([p_ref[0, :N_NODES], p_ref[1, :N_NODES]], axis=1)
    h = jnp.maximum(agg * nd_ref[...] + b1_ref[...], 0.0)
    z = jnp.dot(h * ns_ref[...], w2_ref[...],
                preferred_element_type=jnp.float32)
    fh = z.shape[1] // 2
    o_ref[0] = z[:, :fh]
    o_ref[1] = z[:, fh:]


def _tc1_call(x, ns, w1):
    return pl.pallas_call(
        _tc1_body,
        out_shape=jax.ShapeDtypeStruct((2, N_NODES, w1.shape[1] // 2),
                                       jnp.float32),
    )(x, ns, w1)


def _tc2_call(p, nd, b1, ns, w2):
    return pl.pallas_call(
        _tc2_body,
        out_shape=jax.ShapeDtypeStruct((2, N_NODES, w2.shape[1] // 2),
                                       jnp.float32),
    )(p, nd, b1, ns, w2)


def kernel(in_feat, edge_index, W1, b1, W2, b2):
    ei = edge_index.astype(jnp.int32)
    src = ei[0]
    dst = ei[1]
    srcd = src.reshape(NS, DCH, CK)
    dstd = dst.reshape(NS, DCH, CK)
    srcx, dstx = _prep_idx(src, dst)
    ns_pad, nd_pad = _deg_call()(srcd, dstd)
    ns = ns_pad[:N_NODES].reshape(N_NODES, 1)
    nd = nd_pad[:N_NODES].reshape(N_NODES, 1)
    y1h = _tc1_call(in_feat, ns, W1)
    p1 = _make_gsh(64, 2)(y1h, srcx, dstx)
    y2h = _tc2_call(p1, nd, b1, ns, W2)
    return _make_gsh(32, 1, finalize=True)(y2h, srcx, dstx, nd_pad, b2)
